# baseline (speedup 1.0000x reference)
import functools

import jax
import jax.numpy as jnp
from jax import lax
from jax.experimental import pallas as pl
from jax.experimental.pallas import tpu as pltpu
from jax.experimental.pallas import tpu_sc as plsc

_NC = 2
_NS = 16
_NW = _NC * _NS
_G = 128
_K = 2
_CHUNK = _G * _K


def kernel(token_ids, embedding_matrix):
    batch, seq = token_ids.shape
    num_ids = batch * seq
    vocab, dim = embedding_matrix.shape

    b_per_w = num_ids // _NW
    n_chunks = b_per_w // _CHUNK
    total_chunks = _NW * n_chunks
    ids3 = token_ids.reshape(total_chunks, _K, _G).astype(jnp.int32)
    table128 = jnp.zeros((vocab, 128), jnp.float32).at[:, :dim].set(
        embedding_matrix)

    mesh = plsc.VectorSubcoreMesh(core_axis_name="c", subcore_axis_name="s")

    @functools.partial(
        pl.kernel,
        mesh=mesh,
        out_type=jax.ShapeDtypeStruct((num_ids, 128), jnp.float32),
        scratch_types=[
            pltpu.VMEM((_K, _G), jnp.int32),
            pltpu.VMEM((_K, _G), jnp.int32),
            pltpu.VMEM((_CHUNK, 128), jnp.float32),
            pltpu.VMEM((_CHUNK, 128), jnp.float32),
            pltpu.SemaphoreType.DMA,
            pltpu.SemaphoreType.DMA,
        ],
    )
    def sc_gather(table_hbm, ids_hbm, out_hbm,
                  idx0, idx1, rows0, rows1, sem0, sem1):
        wid = lax.axis_index("s") * _NC + lax.axis_index("c")
        base = wid * n_chunks

        def fire(c, idx_v, rows_v, sem):
            pltpu.sync_copy(ids_hbm.at[base + c], idx_v)
            for j in range(_K):
                pltpu.async_copy(
                    table_hbm.at[idx_v.at[j]],
                    rows_v.at[pl.ds(j * _G, _G)],
                    sem,
                )

        def drain(c, rows_v, sem):
            for j in range(_K):
                pltpu.make_async_copy(
                    table_hbm.at[idx0.at[j]],
                    rows_v.at[pl.ds(j * _G, _G)],
                    sem,
                ).wait()
            pltpu.sync_copy(
                rows_v,
                out_hbm.at[pl.ds((base + c) * _CHUNK, _CHUNK)],
            )

        fire(0, idx0, rows0, sem0)
        fire(1, idx1, rows1, sem1)

        def body(i, carry):
            def even_step():
                drain(i, rows0, sem0)
                pl.when(i + 2 < n_chunks)(
                    lambda: fire(i + 2, idx0, rows0, sem0))

            def odd_step():
                drain(i, rows1, sem1)
                pl.when(i + 2 < n_chunks)(
                    lambda: fire(i + 2, idx1, rows1, sem1))

            lax.cond(i % 2 == 0, even_step, odd_step)
            return carry

        lax.fori_loop(0, n_chunks, body, 0)

    out = sc_gather(table128, ids3)
    return out[:, :dim].reshape(batch, seq, dim)


# speedup vs baseline: 1.3043x; 1.3043x over previous
import functools

import jax
import jax.numpy as jnp
from jax import lax
from jax.experimental import pallas as pl
from jax.experimental.pallas import tpu as pltpu
from jax.experimental.pallas import tpu_sc as plsc

_NC = 2
_NS = 16
_NW = _NC * _NS
_G = 128
_K = 2
_CHUNK = _G * _K


def kernel(token_ids, embedding_matrix):
    batch, seq = token_ids.shape
    num_ids = batch * seq
    vocab, dim = embedding_matrix.shape

    b_per_w = num_ids // _NW
    n_chunks = b_per_w // _CHUNK
    total_chunks = _NW * n_chunks
    ids3 = token_ids.reshape(total_chunks, _K, _G).astype(jnp.int32)
    table128 = jnp.pad(embedding_matrix, ((0, 0), (0, 128 - dim)))

    mesh = plsc.VectorSubcoreMesh(core_axis_name="c", subcore_axis_name="s")

    @functools.partial(
        pl.kernel,
        mesh=mesh,
        out_type=jax.ShapeDtypeStruct((num_ids, 128), jnp.float32),
        scratch_types=[
            pltpu.VMEM((_K, _G), jnp.int32),
            pltpu.VMEM((_K, _G), jnp.int32),
            pltpu.VMEM((_CHUNK, 128), jnp.float32),
            pltpu.VMEM((_CHUNK, 128), jnp.float32),
            pltpu.SemaphoreType.DMA,
            pltpu.SemaphoreType.DMA,
        ],
    )
    def sc_gather(table_hbm, ids_hbm, out_hbm,
                  idx0, idx1, rows0, rows1, sem0, sem1):
        wid = lax.axis_index("s") * _NC + lax.axis_index("c")
        base = wid * n_chunks

        def fire(c, idx_v, rows_v, sem):
            pltpu.sync_copy(ids_hbm.at[base + c], idx_v)
            for j in range(_K):
                pltpu.async_copy(
                    table_hbm.at[idx_v.at[j]],
                    rows_v.at[pl.ds(j * _G, _G)],
                    sem,
                )

        def drain(c, rows_v, sem):
            for j in range(_K):
                pltpu.make_async_copy(
                    table_hbm.at[idx0.at[j]],
                    rows_v.at[pl.ds(j * _G, _G)],
                    sem,
                ).wait()
            pltpu.sync_copy(
                rows_v,
                out_hbm.at[pl.ds((base + c) * _CHUNK, _CHUNK)],
            )

        fire(0, idx0, rows0, sem0)
        fire(1, idx1, rows1, sem1)

        def body(i, carry):
            def even_step():
                drain(i, rows0, sem0)
                pl.when(i + 2 < n_chunks)(
                    lambda: fire(i + 2, idx0, rows0, sem0))

            def odd_step():
                drain(i, rows1, sem1)
                pl.when(i + 2 < n_chunks)(
                    lambda: fire(i + 2, idx1, rows1, sem1))

            lax.cond(i % 2 == 0, even_step, odd_step)
            return carry

        lax.fori_loop(0, n_chunks, body, 0)

    out = sc_gather(table128, ids3)
    return out[:, :dim].reshape(batch, seq, dim)


# speedup vs baseline: 1.4118x; 1.0824x over previous
import functools

import jax
import jax.numpy as jnp
from jax import lax
from jax.experimental import pallas as pl
from jax.experimental.pallas import tpu as pltpu
from jax.experimental.pallas import tpu_sc as plsc

_NC = 2
_NS = 16
_NW = _NC * _NS
_G = 128
_K = 4
_CHUNK = _G * _K


def kernel(token_ids, embedding_matrix):
    batch, seq = token_ids.shape
    num_ids = batch * seq
    vocab, dim = embedding_matrix.shape

    b_per_w = num_ids // _NW
    n_chunks = b_per_w // _CHUNK
    total_chunks = _NW * n_chunks
    ids3 = token_ids.reshape(total_chunks, _K, _G).astype(jnp.int32)

    mesh = plsc.VectorSubcoreMesh(core_axis_name="c", subcore_axis_name="s")

    @functools.partial(
        pl.kernel,
        mesh=mesh,
        out_type=jax.ShapeDtypeStruct((num_ids, 128), jnp.float32),
        compiler_params=pltpu.CompilerParams(use_tc_tiling_on_sc=False),
        scratch_types=[
            pltpu.VMEM((_K, _G), jnp.int32),
            pltpu.VMEM((_K, _G), jnp.int32),
            pltpu.VMEM((_CHUNK, dim), jnp.float32),
            pltpu.VMEM((_CHUNK, dim), jnp.float32),
            pltpu.SemaphoreType.DMA,
            pltpu.SemaphoreType.DMA,
        ],
    )
    def sc_gather(table_hbm, ids_hbm, out_hbm,
                  idx0, idx1, rows0, rows1, sem0, sem1):
        wid = lax.axis_index("s") * _NC + lax.axis_index("c")
        base = wid * n_chunks

        def fire(c, idx_v, rows_v, sem):
            pltpu.sync_copy(ids_hbm.at[base + c], idx_v)
            for j in range(_K):
                pltpu.async_copy(
                    table_hbm.at[idx_v.at[j]],
                    rows_v.at[pl.ds(j * _G, _G)],
                    sem,
                )

        def drain(c, rows_v, sem):
            for j in range(_K):
                pltpu.make_async_copy(
                    table_hbm.at[idx0.at[j]],
                    rows_v.at[pl.ds(j * _G, _G)],
                    sem,
                ).wait()
            pltpu.sync_copy(
                rows_v,
                out_hbm.at[pl.ds((base + c) * _CHUNK, _CHUNK),
                           pl.ds(0, dim)],
            )

        fire(0, idx0, rows0, sem0)
        fire(1, idx1, rows1, sem1)

        def body(i, carry):
            def even_step():
                drain(i, rows0, sem0)
                pl.when(i + 2 < n_chunks)(
                    lambda: fire(i + 2, idx0, rows0, sem0))

            def odd_step():
                drain(i, rows1, sem1)
                pl.when(i + 2 < n_chunks)(
                    lambda: fire(i + 2, idx1, rows1, sem1))

            lax.cond(i % 2 == 0, even_step, odd_step)
            return carry

        lax.fori_loop(0, n_chunks, body, 0)

    out = sc_gather(embedding_matrix, ids3)
    return out[:, :dim].reshape(batch, seq, dim)
